# baseline (speedup 1.0000x reference)
import functools

import jax
import jax.numpy as jnp
from jax import lax
from jax.experimental import pallas as pl
from jax.experimental.pallas import tpu as pltpu
from jax.experimental.pallas import tpu_sc as plsc

B, T, D, H, K, M = 2, 2048, 1024, 16, 3, 32768
DH = D // H
N = B * T
SCALE = D * (H ** 0.5)
HI = lax.Precision.HIGHEST
F32 = jnp.float32
BF16 = jnp.bfloat16
NEG = -1.0e30


def _dot(a, b, dims):
    return lax.dot_general(a, b, (dims, ((), ())), preferred_element_type=F32)


_P_ROWS = 512


def _proj_body(x_ref, w_ref, b_ref, q_ref, k_ref, v_ref, qf_ref):
    y = _dot(x_ref[...], w_ref[...], ((1,), (0,))) + b_ref[...]
    q = y[:, :D]
    k = y[:, D:2 * D]
    v = y[:, 2 * D:]
    qn = jnp.sqrt(jnp.sum(q * q, axis=1, keepdims=True))
    kn = jnp.sqrt(jnp.sum(k * k, axis=1, keepdims=True))
    qd = q / jnp.maximum(qn, 1e-12)
    q_ref[...] = qd.astype(BF16)
    qf_ref[...] = qd
    k_ref[...] = (k / jnp.maximum(kn, 1e-12)).astype(BF16)
    v_ref[...] = v.astype(BF16)


def _proj(x2d, w_cat, b_cat):
    out = jax.ShapeDtypeStruct((N, D), BF16)
    return pl.pallas_call(
        _proj_body,
        grid=(N // _P_ROWS,),
        in_specs=[
            pl.BlockSpec((_P_ROWS, D), lambda i: (i, 0)),
            pl.BlockSpec((D, 3 * D), lambda i: (0, 0)),
            pl.BlockSpec((1, 3 * D), lambda i: (0, 0)),
        ],
        out_specs=[pl.BlockSpec((_P_ROWS, D), lambda i: (i, 0))] * 4,
        out_shape=[out, out, out, jax.ShapeDtypeStruct((N, D), F32)],
    )(x2d, w_cat, b_cat)


_TQ = 256


def _attn_body(q_ref, k_ref, v_ref, o_ref, s_ref):
    i = pl.program_id(1)
    q = q_ref[0] * SCALE
    row = lax.broadcasted_iota(jnp.int32, (_TQ, _TQ), 0)
    col = lax.broadcasted_iota(jnp.int32, (_TQ, _TQ), 1)

    def qk_body(j, m):
        kb = k_ref[0, pl.ds(j * _TQ, _TQ), :]
        s = _dot(q, kb, ((1,), (1,)))
        s = jnp.where((j == i) & (col > row), NEG, s)
        s_ref[:, pl.ds(j * _TQ, _TQ)] = s
        return jnp.maximum(m, jnp.max(s, axis=1, keepdims=True))

    m = lax.fori_loop(0, i + 1, qk_body, jnp.full((_TQ, 1), NEG, F32))

    def z_body(j, z):
        p = jnp.exp(s_ref[:, pl.ds(j * _TQ, _TQ)] - m)
        s_ref[:, pl.ds(j * _TQ, _TQ)] = p
        return z + jnp.sum(p, axis=1, keepdims=True)

    z = lax.fori_loop(0, i + 1, z_body, jnp.zeros((_TQ, 1), F32))

    def pv_body(j, acc):
        pn = (s_ref[:, pl.ds(j * _TQ, _TQ)] / z).astype(BF16)
        return acc + _dot(pn, v_ref[0, pl.ds(j * _TQ, _TQ), :], ((1,), (0,)))

    o_ref[0] = lax.fori_loop(0, i + 1, pv_body, jnp.zeros((_TQ, DH), F32))


def _attn(qh, kh, vh):
    return pl.pallas_call(
        _attn_body,
        grid=(B * H, T // _TQ),
        in_specs=[
            pl.BlockSpec((1, _TQ, DH), lambda bh, i: (bh, i, 0)),
            pl.BlockSpec((1, T, DH), lambda bh, i: (bh, 0, 0)),
            pl.BlockSpec((1, T, DH), lambda bh, i: (bh, 0, 0)),
        ],
        out_specs=pl.BlockSpec((1, _TQ, DH), lambda bh, i: (bh, i, 0)),
        out_shape=jax.ShapeDtypeStruct((B * H, T, DH), F32),
        scratch_shapes=[pltpu.VMEM((_TQ, T), F32)],
    )(qh, kh, vh)


_QB = 512
_MT = 1024


_LN = 128
_RS = 64


def _insert_tile(src_ref, sv_ref, si_ref, base):
    lane = lax.broadcasted_iota(jnp.int32, (_RS, _LN), 1)
    for r in range(0, _QB, _RS):
        rs = slice(r, r + _RS)
        v1 = sv_ref[rs, 0 * _LN:1 * _LN]
        v2 = sv_ref[rs, 1 * _LN:2 * _LN]
        v3 = sv_ref[rs, 2 * _LN:3 * _LN]
        i1 = si_ref[rs, 0 * _LN:1 * _LN]
        i2 = si_ref[rs, 1 * _LN:2 * _LN]
        i3 = si_ref[rs, 2 * _LN:3 * _LN]
        for c in range(_MT // _LN):
            sc = src_ref[rs, c * _LN:(c + 1) * _LN]
            gi = lane + (base + c * _LN)
            g1 = sc > v1
            g2 = sc > v2
            g3 = sc > v3
            v3 = jnp.where(g2, v2, jnp.where(g3, sc, v3))
            i3 = jnp.where(g2, i2, jnp.where(g3, gi, i3))
            v2 = jnp.where(g1, v1, jnp.where(g2, sc, v2))
            i2 = jnp.where(g1, i1, jnp.where(g2, gi, i2))
            v1 = jnp.where(g1, sc, v1)
            i1 = jnp.where(g1, gi, i1)
        sv_ref[rs, 0 * _LN:1 * _LN] = v1
        sv_ref[rs, 1 * _LN:2 * _LN] = v2
        sv_ref[rs, 2 * _LN:3 * _LN] = v3
        si_ref[rs, 0 * _LN:1 * _LN] = i1
        si_ref[rs, 1 * _LN:2 * _LN] = i2
        si_ref[rs, 2 * _LN:3 * _LN] = i3


def _sim_body(q_ref, mk_ref, io_ref, sa_ref, sb_ref, sv_ref, si_ref):
    t = pl.program_id(1)
    nt = M // (2 * _MT)

    @pl.when(t == 0)
    def _():
        sv_ref[...] = jnp.full_like(sv_ref[...], NEG)
        si_ref[...] = jnp.full_like(si_ref[...], -1)
        sb_ref[...] = jnp.full_like(sb_ref[...], NEG)

    sa_ref[...] = _dot(q_ref[...], mk_ref[:_MT, :], ((1,), (1,)))
    _insert_tile(sb_ref, sv_ref, si_ref, (2 * t - 1) * _MT)
    sb_ref[...] = _dot(q_ref[...], mk_ref[_MT:, :], ((1,), (1,)))
    _insert_tile(sa_ref, sv_ref, si_ref, (2 * t) * _MT)

    @pl.when(t == nt - 1)
    def _():
        _insert_tile(sb_ref, sv_ref, si_ref, (2 * t + 1) * _MT)
        a1 = sv_ref[:, 0 * _LN:1 * _LN]
        a2 = sv_ref[:, 1 * _LN:2 * _LN]
        a3 = sv_ref[:, 2 * _LN:3 * _LN]
        b1 = si_ref[:, 0 * _LN:1 * _LN]
        b2 = si_ref[:, 1 * _LN:2 * _LN]
        b3 = si_ref[:, 2 * _LN:3 * _LN]

        def lex_gt(va, ia, vb, ib):
            return (va > vb) | ((va == vb) & (ia < ib))

        for sh in (64, 32, 16, 8, 4, 2, 1):
            c1 = pltpu.roll(a1, sh, 1)
            c2 = pltpu.roll(a2, sh, 1)
            c3 = pltpu.roll(a3, sh, 1)
            d1 = pltpu.roll(b1, sh, 1)
            d2 = pltpu.roll(b2, sh, 1)
            d3 = pltpu.roll(b3, sh, 1)
            for cv, ci in ((c1, d1), (c2, d2), (c3, d3)):
                dup = (ci == b1) | (ci == b2) | (ci == b3)
                t1 = lex_gt(cv, ci, a1, b1) & ~dup
                t2 = lex_gt(cv, ci, a2, b2) & ~dup
                t3 = lex_gt(cv, ci, a3, b3) & ~dup
                a3 = jnp.where(t2, a2, jnp.where(t3, cv, a3))
                b3 = jnp.where(t2, b2, jnp.where(t3, ci, b3))
                a2 = jnp.where(t1, a1, jnp.where(t2, cv, a2))
                b2 = jnp.where(t1, b1, jnp.where(t2, ci, b2))
                a1 = jnp.where(t1, cv, a1)
                b1 = jnp.where(t1, ci, b1)
        io_ref[:, 0:1] = b1[:, 0:1]
        io_ref[:, 1:2] = b2[:, 0:1]
        io_ref[:, 2:3] = b3[:, 0:1]


def _sim_topk(qn_bf16, mem_keys_bf16):
    return pl.pallas_call(
        _sim_body,
        grid=(N // _QB, M // (2 * _MT)),
        in_specs=[
            pl.BlockSpec((_QB, D), lambda qi, t: (qi, 0)),
            pl.BlockSpec((2 * _MT, D), lambda qi, t: (t, 0)),
        ],
        out_specs=pl.BlockSpec((_QB, K), lambda qi, t: (qi, 0)),
        out_shape=jax.ShapeDtypeStruct((N, K), jnp.int32),
        scratch_shapes=[
            pltpu.VMEM((_QB, _MT), F32),
            pltpu.VMEM((_QB, _MT), F32),
            pltpu.VMEM((_QB, K * _LN), F32),
            pltpu.VMEM((_QB, K * _LN), jnp.int32),
        ],
    )(qn_bf16, mem_keys_bf16)


_NW = 32
_TOK_W = N // _NW
_CH = 32


def _gather_sc(mem_keys, mem_values, idx_t):
    mesh = plsc.VectorSubcoreMesh(core_axis_name="c", subcore_axis_name="s")
    out = jax.ShapeDtypeStruct((K, N, D), F32)

    @functools.partial(
        pl.kernel,
        mesh=mesh,
        out_type=[out, out],
        scratch_types=[
            pltpu.VMEM((_CH,), jnp.int32),
            pltpu.VMEM((_CH, D), F32),
            pltpu.VMEM((_CH, D), F32),
            pltpu.SemaphoreType.DMA,
            pltpu.SemaphoreType.DMA,
        ],
    )
    def _gather(mk_hbm, mv_hbm, idx_hbm, gk_hbm, gv_hbm,
                idx_v, rows_k, rows_v, sem_k, sem_v):
        c = lax.axis_index("c")
        sub = lax.axis_index("s")
        w = sub * 2 + c
        for k in range(K):
            for ch in range(_TOK_W // _CH):
                base = w * _TOK_W + ch * _CH
                pltpu.sync_copy(idx_hbm.at[k, pl.ds(base, _CH)], idx_v)
                ck = pltpu.async_copy(mk_hbm.at[idx_v], rows_k, sem_k)
                cv = pltpu.async_copy(mv_hbm.at[idx_v], rows_v, sem_v)
                ck.wait()
                pltpu.sync_copy(rows_k, gk_hbm.at[k, pl.ds(base, _CH)])
                cv.wait()
                pltpu.sync_copy(rows_v, gv_hbm.at[k, pl.ds(base, _CH)])

    return _gather(mem_keys, mem_values, idx_t)


_TB = 256


def _comb_body(q_ref, gk_ref, gv_ref, qkv_ref, g_ref, wo_ref, bo_ref, o_ref):
    q = q_ref[...]
    eh = (lax.broadcasted_iota(jnp.int32, (D, H), 0) // DH
          == lax.broadcasted_iota(jnp.int32, (D, H), 1)).astype(F32)
    ps = jnp.concatenate([q * gk_ref[k] for k in range(K)], axis=0)
    s_all = lax.dot_general(ps, eh, (((1,), (0,)), ((), ())),
                            precision=HI) * SCALE
    s_k = [s_all[k * _TB:(k + 1) * _TB] for k in range(K)]
    m = jnp.maximum(jnp.maximum(s_k[0], s_k[1]), s_k[2])
    e_k = [jnp.exp(s - m) for s in s_k]
    z = e_k[0] + e_k[1] + e_k[2]
    w_all = jnp.concatenate([e / z for e in e_k], axis=0)
    wb_all = lax.dot_general(w_all, eh, (((1,), (1,)), ((), ())),
                             precision=HI)
    mem = jnp.zeros((_TB, D), F32)
    for k in range(K):
        mem = mem + wb_all[k * _TB:(k + 1) * _TB] * gv_ref[k]
    g = g_ref[...]
    comb = (mem * g + qkv_ref[...] * (1.0 - g)).astype(BF16)
    out = _dot(comb, wo_ref[...], ((1,), (0,))) + bo_ref[...]
    o_ref[...] = out


def _combine(qn_f32, gk, gv, qkv2d, g_vec, wo_t_bf16, bo):
    return pl.pallas_call(
        _comb_body,
        grid=(N // _TB,),
        in_specs=[
            pl.BlockSpec((_TB, D), lambda i: (i, 0)),
            pl.BlockSpec((K, _TB, D), lambda i: (0, i, 0)),
            pl.BlockSpec((K, _TB, D), lambda i: (0, i, 0)),
            pl.BlockSpec((_TB, D), lambda i: (i, 0)),
            pl.BlockSpec((1, D), lambda i: (0, 0)),
            pl.BlockSpec((D, D), lambda i: (0, 0)),
            pl.BlockSpec((1, D), lambda i: (0, 0)),
        ],
        out_specs=pl.BlockSpec((_TB, D), lambda i: (i, 0)),
        out_shape=jax.ShapeDtypeStruct((N, D), F32),
    )(qn_f32, gk, gv, qkv2d, g_vec, wo_t_bf16, bo)


def kernel(x, mem_keys, mem_values, Wq, bq, Wk, bk, Wv, bv, Wo, bo, gate_bias):
    x2d = x.reshape(N, D).astype(BF16)
    w_cat = jnp.concatenate([Wq.T, Wk.T, Wv.T], axis=1).astype(BF16)
    b_cat = jnp.concatenate([bq, bk, bv])[None, :]
    qn, kn, v, qn_f32 = _proj(x2d, w_cat, b_cat)

    def heads(a):
        return (a.reshape(B, T, H, DH).transpose(0, 2, 1, 3)
                .reshape(B * H, T, DH))

    idx = _sim_topk(qn, mem_keys.astype(BF16))
    idx_t = idx.T
    gk, gv = _gather_sc(mem_keys, mem_values, idx_t)

    qkv_h = _attn(heads(qn), heads(kn), heads(v))
    qkv2d = (qkv_h.reshape(B, H, T, DH).transpose(0, 2, 1, 3)
             .reshape(N, D))

    g_vec = jnp.repeat(gate_bias.reshape(H), DH)[None, :]
    out2d = _combine(qn_f32, gk, gv, qkv2d, g_vec,
                     Wo.T.astype(BF16), bo[None, :])
    return out2d.reshape(B, T, D)


# speedup vs baseline: 1.2621x; 1.2621x over previous
import functools

import jax
import jax.numpy as jnp
from jax import lax
from jax.experimental import pallas as pl
from jax.experimental.pallas import tpu as pltpu
from jax.experimental.pallas import tpu_sc as plsc

B, T, D, H, K, M = 2, 2048, 1024, 16, 3, 32768
DH = D // H
N = B * T
SCALE = D * (H ** 0.5)
HI = lax.Precision.HIGHEST
F32 = jnp.float32
BF16 = jnp.bfloat16
NEG = -1.0e30


def _dot(a, b, dims):
    return lax.dot_general(a, b, (dims, ((), ())), preferred_element_type=F32)


_P_ROWS = 512


def _proj_body(x_ref, w_ref, b_ref, q_ref, k_ref, v_ref, qf_ref):
    y = _dot(x_ref[...], w_ref[...], ((1,), (0,))) + b_ref[...]
    q = y[:, :D]
    k = y[:, D:2 * D]
    v = y[:, 2 * D:]
    qn = jnp.sqrt(jnp.sum(q * q, axis=1, keepdims=True))
    kn = jnp.sqrt(jnp.sum(k * k, axis=1, keepdims=True))
    qd = q / jnp.maximum(qn, 1e-12)
    q_ref[...] = qd.astype(BF16)
    qf_ref[...] = qd
    k_ref[...] = (k / jnp.maximum(kn, 1e-12)).astype(BF16)
    v_ref[...] = v.astype(BF16)


def _proj(x2d, w_cat, b_cat):
    out = jax.ShapeDtypeStruct((N, D), BF16)
    return pl.pallas_call(
        _proj_body,
        grid=(N // _P_ROWS,),
        in_specs=[
            pl.BlockSpec((_P_ROWS, D), lambda i: (i, 0)),
            pl.BlockSpec((D, 3 * D), lambda i: (0, 0)),
            pl.BlockSpec((1, 3 * D), lambda i: (0, 0)),
        ],
        out_specs=[pl.BlockSpec((_P_ROWS, D), lambda i: (i, 0))] * 4,
        out_shape=[out, out, out, jax.ShapeDtypeStruct((N, D), F32)],
    )(x2d, w_cat, b_cat)


_TQ = 512


def _attn_body(q_ref, k_ref, v_ref, o_ref, s_ref):
    i = pl.program_id(1)
    q = q_ref[0] * SCALE
    row = lax.broadcasted_iota(jnp.int32, (_TQ, _TQ), 0)
    col = lax.broadcasted_iota(jnp.int32, (_TQ, _TQ), 1)

    def qk_body(j, m):
        kb = k_ref[0, pl.ds(j * _TQ, _TQ), :]
        s = _dot(q, kb, ((1,), (1,)))
        s = jnp.where((j == i) & (col > row), NEG, s)
        s_ref[:, pl.ds(j * _TQ, _TQ)] = s
        return jnp.maximum(m, jnp.max(s, axis=1, keepdims=True))

    m = lax.fori_loop(0, i + 1, qk_body, jnp.full((_TQ, 1), NEG, F32))

    def z_body(j, z):
        p = jnp.exp(s_ref[:, pl.ds(j * _TQ, _TQ)] - m)
        s_ref[:, pl.ds(j * _TQ, _TQ)] = p
        return z + jnp.sum(p, axis=1, keepdims=True)

    z = lax.fori_loop(0, i + 1, z_body, jnp.zeros((_TQ, 1), F32))

    def pv_body(j, acc):
        pn = (s_ref[:, pl.ds(j * _TQ, _TQ)] / z).astype(BF16)
        return acc + _dot(pn, v_ref[0, pl.ds(j * _TQ, _TQ), :], ((1,), (0,)))

    o_ref[0] = lax.fori_loop(0, i + 1, pv_body, jnp.zeros((_TQ, DH), F32))


def _attn(qh, kh, vh):
    return pl.pallas_call(
        _attn_body,
        grid=(B * H, T // _TQ),
        in_specs=[
            pl.BlockSpec((1, _TQ, DH), lambda bh, i: (bh, i, 0)),
            pl.BlockSpec((1, T, DH), lambda bh, i: (bh, 0, 0)),
            pl.BlockSpec((1, T, DH), lambda bh, i: (bh, 0, 0)),
        ],
        out_specs=pl.BlockSpec((1, _TQ, DH), lambda bh, i: (bh, i, 0)),
        out_shape=jax.ShapeDtypeStruct((B * H, T, DH), F32),
        scratch_shapes=[pltpu.VMEM((_TQ, T), F32)],
    )(qh, kh, vh)


_QB = 512
_MT = 1024


_LN = 128
_RS = 64


def _insert_tile(src_ref, sv_ref, si_ref, base):
    lane = lax.broadcasted_iota(jnp.int32, (_RS, _LN), 1)
    for r in range(0, _QB, _RS):
        rs = slice(r, r + _RS)
        v1 = sv_ref[rs, 0 * _LN:1 * _LN]
        v2 = sv_ref[rs, 1 * _LN:2 * _LN]
        v3 = sv_ref[rs, 2 * _LN:3 * _LN]
        i1 = si_ref[rs, 0 * _LN:1 * _LN]
        i2 = si_ref[rs, 1 * _LN:2 * _LN]
        i3 = si_ref[rs, 2 * _LN:3 * _LN]
        for c in range(_MT // _LN):
            sc = src_ref[rs, c * _LN:(c + 1) * _LN]
            gi = lane + (base + c * _LN)
            g1 = sc > v1
            g2 = sc > v2
            g3 = sc > v3
            v3 = jnp.where(g2, v2, jnp.where(g3, sc, v3))
            i3 = jnp.where(g2, i2, jnp.where(g3, gi, i3))
            v2 = jnp.where(g1, v1, jnp.where(g2, sc, v2))
            i2 = jnp.where(g1, i1, jnp.where(g2, gi, i2))
            v1 = jnp.where(g1, sc, v1)
            i1 = jnp.where(g1, gi, i1)
        sv_ref[rs, 0 * _LN:1 * _LN] = v1
        sv_ref[rs, 1 * _LN:2 * _LN] = v2
        sv_ref[rs, 2 * _LN:3 * _LN] = v3
        si_ref[rs, 0 * _LN:1 * _LN] = i1
        si_ref[rs, 1 * _LN:2 * _LN] = i2
        si_ref[rs, 2 * _LN:3 * _LN] = i3


def _sim_body(q_ref, mk_ref, io_ref, sa_ref, sb_ref, sv_ref, si_ref):
    t = pl.program_id(1)
    nt = M // (2 * _MT)

    @pl.when(t == 0)
    def _():
        sv_ref[...] = jnp.full_like(sv_ref[...], NEG)
        si_ref[...] = jnp.full_like(si_ref[...], -1)
        sb_ref[...] = jnp.full_like(sb_ref[...], NEG)

    sa_ref[...] = _dot(q_ref[...], mk_ref[:_MT, :], ((1,), (1,)))
    _insert_tile(sb_ref, sv_ref, si_ref, (2 * t - 1) * _MT)
    sb_ref[...] = _dot(q_ref[...], mk_ref[_MT:, :], ((1,), (1,)))
    _insert_tile(sa_ref, sv_ref, si_ref, (2 * t) * _MT)

    @pl.when(t == nt - 1)
    def _():
        _insert_tile(sb_ref, sv_ref, si_ref, (2 * t + 1) * _MT)
        a1 = sv_ref[:, 0 * _LN:1 * _LN]
        a2 = sv_ref[:, 1 * _LN:2 * _LN]
        a3 = sv_ref[:, 2 * _LN:3 * _LN]
        b1 = si_ref[:, 0 * _LN:1 * _LN]
        b2 = si_ref[:, 1 * _LN:2 * _LN]
        b3 = si_ref[:, 2 * _LN:3 * _LN]

        def lex_gt(va, ia, vb, ib):
            return (va > vb) | ((va == vb) & (ia < ib))

        for sh in (64, 32, 16, 8, 4, 2, 1):
            c1 = pltpu.roll(a1, sh, 1)
            c2 = pltpu.roll(a2, sh, 1)
            c3 = pltpu.roll(a3, sh, 1)
            d1 = pltpu.roll(b1, sh, 1)
            d2 = pltpu.roll(b2, sh, 1)
            d3 = pltpu.roll(b3, sh, 1)
            for cv, ci in ((c1, d1), (c2, d2), (c3, d3)):
                dup = (ci == b1) | (ci == b2) | (ci == b3)
                t1 = lex_gt(cv, ci, a1, b1) & ~dup
                t2 = lex_gt(cv, ci, a2, b2) & ~dup
                t3 = lex_gt(cv, ci, a3, b3) & ~dup
                a3 = jnp.where(t2, a2, jnp.where(t3, cv, a3))
                b3 = jnp.where(t2, b2, jnp.where(t3, ci, b3))
                a2 = jnp.where(t1, a1, jnp.where(t2, cv, a2))
                b2 = jnp.where(t1, b1, jnp.where(t2, ci, b2))
                a1 = jnp.where(t1, cv, a1)
                b1 = jnp.where(t1, ci, b1)
        io_ref[:, 0:1] = b1[:, 0:1]
        io_ref[:, 1:2] = b2[:, 0:1]
        io_ref[:, 2:3] = b3[:, 0:1]


def _sim_topk(qn_bf16, mem_keys_bf16):
    return pl.pallas_call(
        _sim_body,
        grid=(N // _QB, M // (2 * _MT)),
        in_specs=[
            pl.BlockSpec((_QB, D), lambda qi, t: (qi, 0)),
            pl.BlockSpec((2 * _MT, D), lambda qi, t: (t, 0)),
        ],
        out_specs=pl.BlockSpec((_QB, K), lambda qi, t: (qi, 0)),
        out_shape=jax.ShapeDtypeStruct((N, K), jnp.int32),
        scratch_shapes=[
            pltpu.VMEM((_QB, _MT), F32),
            pltpu.VMEM((_QB, _MT), F32),
            pltpu.VMEM((_QB, K * _LN), F32),
            pltpu.VMEM((_QB, K * _LN), jnp.int32),
        ],
    )(qn_bf16, mem_keys_bf16)


_NW = 32
_TOK_W = N // _NW
_CH = 32


def _gather_sc(mem_keys, mem_values, idx_t):
    mesh = plsc.VectorSubcoreMesh(core_axis_name="c", subcore_axis_name="s")
    out = jax.ShapeDtypeStruct((K, N, D), F32)

    @functools.partial(
        pl.kernel,
        mesh=mesh,
        out_type=[out, out],
        scratch_types=[
            pltpu.VMEM((_CH,), jnp.int32),
            pltpu.VMEM((_CH, D), F32),
            pltpu.VMEM((_CH, D), F32),
            pltpu.SemaphoreType.DMA,
            pltpu.SemaphoreType.DMA,
        ],
    )
    def _gather(mk_hbm, mv_hbm, idx_hbm, gk_hbm, gv_hbm,
                idx_v, rows_k, rows_v, sem_k, sem_v):
        c = lax.axis_index("c")
        sub = lax.axis_index("s")
        w = sub * 2 + c
        for k in range(K):
            for ch in range(_TOK_W // _CH):
                base = w * _TOK_W + ch * _CH
                pltpu.sync_copy(idx_hbm.at[k, pl.ds(base, _CH)], idx_v)
                ck = pltpu.async_copy(mk_hbm.at[idx_v], rows_k, sem_k)
                cv = pltpu.async_copy(mv_hbm.at[idx_v], rows_v, sem_v)
                ck.wait()
                pltpu.sync_copy(rows_k, gk_hbm.at[k, pl.ds(base, _CH)])
                cv.wait()
                pltpu.sync_copy(rows_v, gv_hbm.at[k, pl.ds(base, _CH)])

    return _gather(mem_keys, mem_values, idx_t)


_TB = 512


def _comb_body(q_ref, gk_ref, gv_ref, qkv_ref, g_ref, wo_ref, bo_ref, o_ref):
    q = q_ref[...]
    eh = (lax.broadcasted_iota(jnp.int32, (D, H), 0) // DH
          == lax.broadcasted_iota(jnp.int32, (D, H), 1)).astype(F32)
    ps = jnp.concatenate([q * gk_ref[k] for k in range(K)], axis=0)
    s_all = lax.dot_general(ps, eh, (((1,), (0,)), ((), ())),
                            precision=HI) * SCALE
    s_k = [s_all[k * _TB:(k + 1) * _TB] for k in range(K)]
    m = jnp.maximum(jnp.maximum(s_k[0], s_k[1]), s_k[2])
    e_k = [jnp.exp(s - m) for s in s_k]
    z = e_k[0] + e_k[1] + e_k[2]
    w_all = jnp.concatenate([e / z for e in e_k], axis=0)
    wb_all = lax.dot_general(w_all, eh, (((1,), (1,)), ((), ())),
                             precision=HI)
    mem = jnp.zeros((_TB, D), F32)
    for k in range(K):
        mem = mem + wb_all[k * _TB:(k + 1) * _TB] * gv_ref[k]
    g = g_ref[...]
    comb = (mem * g + qkv_ref[...] * (1.0 - g)).astype(BF16)
    out = _dot(comb, wo_ref[...], ((1,), (0,))) + bo_ref[...]
    o_ref[...] = out


def _combine(qn_f32, gk, gv, qkv2d, g_vec, wo_t_bf16, bo):
    return pl.pallas_call(
        _comb_body,
        grid=(N // _TB,),
        in_specs=[
            pl.BlockSpec((_TB, D), lambda i: (i, 0)),
            pl.BlockSpec((K, _TB, D), lambda i: (0, i, 0)),
            pl.BlockSpec((K, _TB, D), lambda i: (0, i, 0)),
            pl.BlockSpec((_TB, D), lambda i: (i, 0)),
            pl.BlockSpec((1, D), lambda i: (0, 0)),
            pl.BlockSpec((D, D), lambda i: (0, 0)),
            pl.BlockSpec((1, D), lambda i: (0, 0)),
        ],
        out_specs=pl.BlockSpec((_TB, D), lambda i: (i, 0)),
        out_shape=jax.ShapeDtypeStruct((N, D), F32),
    )(qn_f32, gk, gv, qkv2d, g_vec, wo_t_bf16, bo)


def kernel(x, mem_keys, mem_values, Wq, bq, Wk, bk, Wv, bv, Wo, bo, gate_bias):
    x2d = x.reshape(N, D).astype(BF16)
    w_cat = jnp.concatenate([Wq.T, Wk.T, Wv.T], axis=1).astype(BF16)
    b_cat = jnp.concatenate([bq, bk, bv])[None, :]
    qn, kn, v, qn_f32 = _proj(x2d, w_cat, b_cat)

    def heads(a):
        return (a.reshape(B, T, H, DH).transpose(0, 2, 1, 3)
                .reshape(B * H, T, DH))

    idx = _sim_topk(qn, mem_keys.astype(BF16))
    idx_t = idx.T
    gk, gv = _gather_sc(mem_keys, mem_values, idx_t)

    qkv_h = _attn(heads(qn), heads(kn), heads(v))
    qkv2d = (qkv_h.reshape(B, H, T, DH).transpose(0, 2, 1, 3)
             .reshape(N, D))

    g_vec = jnp.repeat(gate_bias.reshape(H), DH)[None, :]
    out2d = _combine(qn_f32, gk, gv, qkv2d, g_vec,
                     Wo.T.astype(BF16), bo[None, :])
    return out2d.reshape(B, T, D)
